# Initial kernel scaffold; baseline (speedup 1.0000x reference)
import jax
import jax.numpy as jnp
from jax.experimental import pallas as pl


def kernel(embed_table, gate_W, gate_b, diag_table, pro_table, med2diag, med2pro, ehradj, viewcat_W, viewcat_b, sel_W, sel_b, avg_proj, fingerprints, edge_src, edge_dst, seg_ids):
    raise NotImplementedError("write your pallas kernel here")


# speedup vs baseline: 6.1433x; 6.1433x over previous
import functools

import jax
import jax.numpy as jnp
from jax import lax
from jax.experimental import pallas as pl
from jax.experimental.pallas import tpu as pltpu
from jax.experimental.pallas import tpu_sc as plsc

N_NODES = 10000
N_EDGES = 160000
DIM = 256
HALF = 128
N_FP = 10000
N_MOL = 500
N_MED = 150
EPS = 0.3
LAYER_NUM = 2

NPAD = 10240
EPAD = 163840
SEGP = 512
NT = 640
ET = 5120
CH = 128
NCHUNK_N = NT // CH
NCHUNK_E = ET // CH
ECORE = EPAD // 2

_MESH = dict(core_axis_name="c", subcore_axis_name="s")


def _iota16():
    return lax.broadcasted_iota(jnp.int32, (16,), 0)


def _sc_tanh(x):
    ax = jnp.abs(x)
    t = 1.0 - 2.0 / (jnp.exp(2.0 * ax) + 1.0)
    return jnp.where(x < 0.0, -t, t)


def _init_body(fp_hbm, emb_hbm, edst_hbm, h0_hbm, degp_hbm,
               idx_v, adj_v, rows_v, ones_v, dst_v, buf16_v, deg_v, acc_sh, sem):
    c = lax.axis_index("c")
    s = lax.axis_index("s")
    base_n = s * NT

    def zrow(j, _):
        buf16_v[j, :] = jnp.zeros((16,), jnp.float32)
        return 0
    lax.fori_loop(0, NT, zrow, 0)
    pltpu.sync_copy(buf16_v, acc_sh.at[pl.ds(base_n, NT)])

    e0 = (_iota16() == 0).astype(jnp.float32)
    def orow(j, _):
        ones_v[j, :] = e0
        return 0
    lax.fori_loop(0, CH, orow, 0)
    plsc.subcore_barrier()

    for k in range(NCHUNK_N):
        nb = base_n + k * CH
        pltpu.sync_copy(fp_hbm.at[pl.ds(nb, CH)], idx_v)
        for q in range(CH // 16):
            sl = pl.ds(q * 16, 16)
            adj_v[sl] = idx_v[sl] + c * N_FP
        pltpu.async_copy(emb_hbm.at[adj_v], rows_v, sem).wait()
        pltpu.sync_copy(rows_v, h0_hbm.at[pl.ds(c * NPAD + nb, CH)])

    def deg_step(k, _):
        eb = c * ECORE + s * ET + k * CH
        pltpu.sync_copy(edst_hbm.at[pl.ds(eb, CH)], dst_v)
        pltpu.sync_copy(ones_v, acc_sh.at[dst_v], add=True)
        return 0
    lax.fori_loop(0, NCHUNK_E, deg_step, 0)
    plsc.subcore_barrier()

    pltpu.sync_copy(acc_sh.at[pl.ds(base_n, NT)], buf16_v)
    z16 = jnp.zeros((16,), jnp.int32)
    def ext(k, _):
        ridx = jnp.full((16,), k * 16, jnp.int32) + _iota16()
        vals = plsc.load_gather(buf16_v, [ridx, z16])
        deg_v[pl.ds(k * 16, 16)] = vals
        return 0
    lax.fori_loop(0, NT // 16, ext, 0)
    pltpu.sync_copy(deg_v, degp_hbm.at[c, pl.ds(base_n, NT)])


def _sc_init(fp_p, emb_cat, edst_p):
    return pl.kernel(
        _init_body,
        out_type=(
            jax.ShapeDtypeStruct((2 * NPAD, HALF), jnp.float32),
            jax.ShapeDtypeStruct((2, NPAD), jnp.float32),
        ),
        mesh=plsc.VectorSubcoreMesh(**_MESH),
        compiler_params=pltpu.CompilerParams(
            needs_layout_passes=False, use_tc_tiling_on_sc=False),
        scratch_types=[
            pltpu.VMEM((CH,), jnp.int32),
            pltpu.VMEM((CH,), jnp.int32),
            pltpu.VMEM((CH, HALF), jnp.float32),
            pltpu.VMEM((CH, 16), jnp.float32),
            pltpu.VMEM((CH,), jnp.int32),
            pltpu.VMEM((NT, 16), jnp.float32),
            pltpu.VMEM((NT,), jnp.float32),
            pltpu.VMEM_SHARED((NPAD, 16), jnp.float32),
            pltpu.SemaphoreType.DMA,
        ],
    )(fp_p, emb_cat, edst_p)


def _edge_body(pool, h_hbm, p_hbm, esrc_hbm, edst_hbm, seg_hbm, *refs):
    (out_hbm, src_v, dst_v, adj_v, e_v, rows_v, mbuf_v, pd_v, ps_v,
     seg_v, acc_sh, mol_sh, sem, semp) = refs
    c = lax.axis_index("c")
    s = lax.axis_index("s")
    base_n = s * NT

    def zrow(j, _):
        for q in range(HALF // 16):
            mbuf_v[j, pl.ds(q * 16, 16)] = jnp.zeros((16,), jnp.float32)
        return 0
    lax.fori_loop(0, CH, zrow, 0)
    for k in range(NCHUNK_N):
        pltpu.sync_copy(mbuf_v, acc_sh.at[pl.ds(base_n + k * CH, CH)])
    if pool:
        @pl.when(s < 4)
        def _():
            pltpu.sync_copy(mbuf_v, mol_sh.at[pl.ds(s * CH, CH)])
    plsc.subcore_barrier()

    def edge_step(k, _):
        eb = c * ECORE + s * ET + k * CH
        pltpu.sync_copy(esrc_hbm.at[pl.ds(eb, CH)], src_v)
        pltpu.sync_copy(edst_hbm.at[pl.ds(eb, CH)], dst_v)
        for q in range(CH // 16):
            sl = pl.ds(q * 16, 16)
            adj_v[sl] = src_v[sl] + c * NPAD
        cp_rows = pltpu.async_copy(h_hbm.at[adj_v], rows_v, sem)
        pltpu.async_copy(p_hbm.at[dst_v], pd_v, semp).wait()
        pltpu.async_copy(p_hbm.at[src_v], ps_v, semp).wait()
        iota = _iota16()
        c0 = jnp.zeros((16,), jnp.int32)
        c1 = jnp.full((16,), 1, jnp.int32)
        c2 = jnp.full((16,), 2, jnp.int32)
        for q in range(CH // 16):
            r16 = jnp.full((16,), q * 16, jnp.int32) + iota
            ud = plsc.load_gather(pd_v, [r16, c0])
            nd = plsc.load_gather(pd_v, [r16, c2])
            vs = plsc.load_gather(ps_v, [r16, c1])
            ns = plsc.load_gather(ps_v, [r16, c2])
            e_v[pl.ds(q * 16, 16)] = _sc_tanh(ud + vs) * nd * ns
        cp_rows.wait()

        def scale(j, _):
            bc = plsc.load_gather(e_v, [jnp.full((16,), j, jnp.int32)])
            for q in range(HALF // 16):
                sl = pl.ds(q * 16, 16)
                rows_v[j, sl] = rows_v[j, sl] * bc
            return 0
        lax.fori_loop(0, CH, scale, 0)
        pltpu.sync_copy(rows_v, acc_sh.at[dst_v], add=True)
        return 0
    lax.fori_loop(0, NCHUNK_E, edge_step, 0)
    plsc.subcore_barrier()

    for k in range(NCHUNK_N):
        nb = base_n + k * CH
        pltpu.sync_copy(h_hbm.at[pl.ds(c * NPAD + nb, CH)], rows_v)
        pltpu.sync_copy(acc_sh.at[pl.ds(nb, CH)], mbuf_v)

        def upd(j, _):
            for q in range(HALF // 16):
                sl = pl.ds(q * 16, 16)
                rows_v[j, sl] = jnp.maximum(
                    EPS * rows_v[j, sl] + mbuf_v[j, sl], 0.0)
            return 0
        lax.fori_loop(0, CH, upd, 0)
        if pool:
            pltpu.sync_copy(seg_hbm.at[pl.ds(nb, CH)], seg_v)
            pltpu.sync_copy(rows_v, mol_sh.at[seg_v], add=True)
        else:
            pltpu.sync_copy(rows_v, out_hbm.at[pl.ds(c * NPAD + nb, CH)])

    if pool:
        plsc.subcore_barrier()
        @pl.when(s < 4)
        def _():
            pltpu.sync_copy(mol_sh.at[pl.ds(s * CH, CH)], rows_v)
            pltpu.sync_copy(rows_v, out_hbm.at[c, pl.ds(s * CH, CH)])


def _sc_edge(pool, h, ptab, esrc_p, edst_p, seg_p):
    if pool:
        out_type = jax.ShapeDtypeStruct((2, SEGP, HALF), jnp.float32)
    else:
        out_type = jax.ShapeDtypeStruct((2 * NPAD, HALF), jnp.float32)
    return pl.kernel(
        functools.partial(_edge_body, pool),
        out_type=out_type,
        mesh=plsc.VectorSubcoreMesh(**_MESH),
        compiler_params=pltpu.CompilerParams(
            needs_layout_passes=False, use_tc_tiling_on_sc=False),
        scratch_types=[
            pltpu.VMEM((CH,), jnp.int32),
            pltpu.VMEM((CH,), jnp.int32),
            pltpu.VMEM((CH,), jnp.int32),
            pltpu.VMEM((CH,), jnp.float32),
            pltpu.VMEM((CH, HALF), jnp.float32),
            pltpu.VMEM((CH, HALF), jnp.float32),
            pltpu.VMEM((CH, 4), jnp.float32),
            pltpu.VMEM((CH, 4), jnp.float32),
            pltpu.VMEM((CH,), jnp.int32),
            pltpu.VMEM_SHARED((NPAD, HALF), jnp.float32),
            pltpu.VMEM_SHARED((SEGP, HALF), jnp.float32),
            pltpu.SemaphoreType.DMA,
            pltpu.SemaphoreType.DMA,
        ],
    )(h, ptab, esrc_p, edst_p, seg_p)


def _uv_body(h_ref, w_ref, b_ref, n_ref, out_ref):
    c = pl.program_id(1)
    blk = h_ref.shape[0]
    part = jnp.dot(h_ref[...], w_ref[0],
                   preferred_element_type=jnp.float32)
    zz = jnp.zeros((blk, 2), jnp.float32)

    @pl.when(c == 0)
    def _():
        out_ref[...] = jnp.concatenate(
            [part, n_ref[...], jnp.zeros((blk, 1), jnp.float32)], axis=1)

    @pl.when(c == 1)
    def _():
        out_ref[...] = (out_ref[...]
                        + jnp.concatenate([part, zz], axis=1) + b_ref[...])


def _tc_uv(h, w2, bias4, nrm):
    blk = 1024
    return pl.pallas_call(
        _uv_body,
        grid=(NPAD // blk, 2),
        in_specs=[
            pl.BlockSpec((blk, HALF), lambda i, c: (c * (NPAD // blk) + i, 0)),
            pl.BlockSpec((1, HALF, 2), lambda i, c: (c, 0, 0)),
            pl.BlockSpec((1, 4), lambda i, c: (0, 0)),
            pl.BlockSpec((blk, 1), lambda i, c: (i, 0)),
        ],
        out_specs=pl.BlockSpec((blk, 4), lambda i, c: (i, 0)),
        out_shape=jax.ShapeDtypeStruct((NPAD, 4), jnp.float32),
    )(h, w2, bias4, nrm)


def _norm_body(degp_ref, out_ref):
    d = degp_ref[0] + degp_ref[1]
    out_ref[...] = lax.rsqrt(jnp.maximum(d, 1.0))


def _tc_norm(degp):
    return pl.pallas_call(
        _norm_body,
        out_shape=jax.ShapeDtypeStruct((NPAD // HALF, HALF), jnp.float32),
    )(degp.reshape(2, NPAD // HALF, HALF)).reshape(NPAD)


def _final_body(dt_ref, pt_ref, md_ref, mp_ref, ea_ref, vc1_ref, vc2_ref,
                vcb_ref, sw_ref, sb_ref, ap_ref, mol_ref, out_ref):
    f32 = jnp.float32
    dv = jnp.dot(md_ref[...], dt_ref[...], preferred_element_type=f32)
    pv = jnp.dot(mp_ref[...], pt_ref[...], preferred_element_type=f32)
    rec = (jnp.dot(dv, vc1_ref[...], preferred_element_type=f32)
           + jnp.dot(pv, vc2_ref[...], preferred_element_type=f32)
           + vcb_ref[...])
    aug = jnp.dot(ea_ref[...], rec, preferred_element_type=f32)
    sel = jnp.tanh(jnp.dot(rec, sw_ref[...], preferred_element_type=f32)
                   + sb_ref[...])
    ctx = rec + sel * aug
    out_ref[...] = jnp.dot(ap_ref[...], mol_ref[...],
                           preferred_element_type=f32) + ctx


def _tc_final(diag_table, pro_table, med2diag, med2pro, ehradj, vc1, vc2,
              vcb, sel_W, sel_b, avg_proj, mol):
    return pl.pallas_call(
        _final_body,
        out_shape=jax.ShapeDtypeStruct((N_MED, DIM), jnp.float32),
    )(diag_table, pro_table, med2diag, med2pro, ehradj, vc1, vc2,
      vcb, sel_W, sel_b, avg_proj, mol)


def kernel(embed_table, gate_W, gate_b, diag_table, pro_table, med2diag,
           med2pro, ehradj, viewcat_W, viewcat_b, sel_W, sel_b, avg_proj,
           fingerprints, edge_src, edge_dst, seg_ids):
    i32 = jnp.int32
    emb_cat = jnp.concatenate([embed_table[:, :HALF], embed_table[:, HALF:]], 0)

    fp_p = jnp.concatenate(
        [fingerprints.astype(i32), jnp.zeros((NPAD - N_NODES,), i32)])
    esrc_p = jnp.concatenate(
        [edge_src.astype(i32), jnp.zeros((EPAD - N_EDGES,), i32)])
    edst_p = jnp.concatenate(
        [edge_dst.astype(i32), jnp.full((EPAD - N_EDGES,), N_NODES, i32)])
    seg_p = jnp.concatenate(
        [seg_ids.astype(i32), jnp.full((NPAD - N_NODES,), SEGP - 1, i32)])

    h, degp = _sc_init(fp_p, emb_cat, edst_p)
    nrm = _tc_norm(degp).reshape(NPAD, 1)

    mol2 = None
    z = jnp.zeros((), jnp.float32)
    for l in range(LAYER_NUM):
        wd = gate_W[l, :, 0]
        w2 = jnp.stack(
            [jnp.stack([wd[:HALF], wd[DIM:DIM + HALF]], axis=1),
             jnp.stack([wd[HALF:DIM], wd[DIM + HALF:]], axis=1)], axis=0)
        bias4 = jnp.stack([jnp.stack([gate_b[l, 0], z, z, z])])
        ptab = _tc_uv(h, w2, bias4, nrm)
        if l < LAYER_NUM - 1:
            h = _sc_edge(False, h, ptab, esrc_p, edst_p, seg_p)
        else:
            mol2 = _sc_edge(True, h, ptab, esrc_p, edst_p, seg_p)

    mol = jnp.concatenate([mol2[0, :N_MOL, :], mol2[1, :N_MOL, :]], axis=1)
    vc1 = viewcat_W[:DIM, :]
    vc2 = viewcat_W[DIM:, :]
    return _tc_final(diag_table, pro_table, med2diag, med2pro, ehradj,
                     vc1, vc2, viewcat_b[None, :], sel_W, sel_b[None, :],
                     avg_proj, mol)
